# baseline (speedup 1.0000x reference)
import jax
import jax.numpy as jnp
from jax.experimental import pallas as pl
from jax.experimental.pallas import tpu as pltpu

_N = 21
_R = 64


def _cm_kernel(p_ref, t_ref, out_ref):
    ji = pl.program_id(1)

    x = p_ref[0]
    t = t_ref[0]
    iota = jax.lax.broadcasted_iota(jnp.int32, (_N, 1, 1), 0)
    maxv = jnp.max(x, axis=0, keepdims=True)
    b_oh = (x == maxv).astype(jnp.bfloat16)
    a_oh = (t[None] == iota).astype(jnp.bfloat16)
    cb = jax.lax.dot_general(
        a_oh, b_oh, (((2,), (2,)), ((1,), (1,))),
        preferred_element_type=jnp.float32)
    c = jnp.sum(cb, axis=0)

    @pl.when(ji == 0)
    def _first():
        out_ref[0] = c

    @pl.when(ji != 0)
    def _rest():
        out_ref[0] += c


def _iou_eval_kernel(m_ref, out_ref):
    h = jnp.sum(m_ref[...], axis=0)
    r = jax.lax.broadcasted_iota(jnp.int32, (_N, _N), 0)
    cidx = jax.lax.broadcasted_iota(jnp.int32, (_N, _N), 1)
    eye = (r == cidx).astype(jnp.float32)
    ones = jnp.ones((_N, 1), jnp.float32)
    diag = jax.lax.dot_general(
        h * eye, ones, (((1,), (0,)), ((), ())),
        preferred_element_type=jnp.float32)
    rows = jax.lax.dot_general(
        h, ones, (((1,), (0,)), ((), ())),
        preferred_element_type=jnp.float32)
    cols = jax.lax.dot_general(
        h, ones, (((0,), (0,)), ((), ())),
        preferred_element_type=jnp.float32)
    iou = diag / (rows + cols - diag)
    out_ref[...] = (jnp.sum(iou) / _N).reshape(1, 1)


def kernel(preds, targets, mat):
    batch, n, hh, ww = preds.shape
    nb = hh // _R
    mats = pl.pallas_call(
        _cm_kernel,
        grid=(batch, nb),
        in_specs=[
            pl.BlockSpec((1, n, _R, ww), lambda b, j: (b, 0, j, 0)),
            pl.BlockSpec((1, _R, ww), lambda b, j: (b, j, 0)),
        ],
        out_specs=pl.BlockSpec((1, n, n), lambda b, j: (b, 0, 0)),
        out_shape=jax.ShapeDtypeStruct((batch, n, n), jnp.float32),
        compiler_params=pltpu.CompilerParams(
            dimension_semantics=("parallel", "arbitrary")),
    )(preds, targets)
    out = pl.pallas_call(
        _iou_eval_kernel,
        out_shape=jax.ShapeDtypeStruct((1, 1), jnp.float32),
    )(mats)
    return out[0, 0]


# speedup vs baseline: 3.3872x; 3.3872x over previous
import jax
import jax.numpy as jnp
from jax.experimental import pallas as pl
from jax.experimental.pallas import tpu as pltpu

_N = 21
_R = 64


def _cm_kernel(p_ref, t_ref, out_ref):
    ji = pl.program_id(1)

    x = p_ref[0].reshape(_N, _R * 512)
    t = t_ref[0].reshape(1, _R * 512)
    iota = jax.lax.broadcasted_iota(jnp.int32, (_N, 1), 0)
    maxv = jnp.max(x, axis=0, keepdims=True)
    b_oh = (x == maxv).astype(jnp.bfloat16)
    a_oh = (t == iota).astype(jnp.bfloat16)
    c = jax.lax.dot_general(
        a_oh, b_oh, (((1,), (1,)), ((), ())),
        preferred_element_type=jnp.float32)

    @pl.when(ji == 0)
    def _first():
        out_ref[0] = c

    @pl.when(ji != 0)
    def _rest():
        out_ref[0] += c


def _iou_eval_kernel(m_ref, out_ref):
    h = jnp.sum(m_ref[...], axis=0)
    r = jax.lax.broadcasted_iota(jnp.int32, (_N, _N), 0)
    cidx = jax.lax.broadcasted_iota(jnp.int32, (_N, _N), 1)
    eye = (r == cidx).astype(jnp.float32)
    ones = jnp.ones((_N, 1), jnp.float32)
    diag = jax.lax.dot_general(
        h * eye, ones, (((1,), (0,)), ((), ())),
        preferred_element_type=jnp.float32)
    rows = jax.lax.dot_general(
        h, ones, (((1,), (0,)), ((), ())),
        preferred_element_type=jnp.float32)
    cols = jax.lax.dot_general(
        h, ones, (((0,), (0,)), ((), ())),
        preferred_element_type=jnp.float32)
    iou = diag / (rows + cols - diag)
    out_ref[...] = (jnp.sum(iou) / _N).reshape(1, 1)


def kernel(preds, targets, mat):
    batch, n, hh, ww = preds.shape
    nb = hh // _R
    mats = pl.pallas_call(
        _cm_kernel,
        grid=(batch, nb),
        in_specs=[
            pl.BlockSpec((1, n, _R, ww), lambda b, j: (b, 0, j, 0)),
            pl.BlockSpec((1, _R, ww), lambda b, j: (b, j, 0)),
        ],
        out_specs=pl.BlockSpec((1, n, n), lambda b, j: (b, 0, 0)),
        out_shape=jax.ShapeDtypeStruct((batch, n, n), jnp.float32),
        compiler_params=pltpu.CompilerParams(
            dimension_semantics=("parallel", "arbitrary")),
    )(preds, targets)
    out = pl.pallas_call(
        _iou_eval_kernel,
        out_shape=jax.ShapeDtypeStruct((1, 1), jnp.float32),
    )(mats)
    return out[0, 0]


# speedup vs baseline: 3.6651x; 1.0820x over previous
import jax
import jax.numpy as jnp
from jax.experimental import pallas as pl
from jax.experimental.pallas import tpu as pltpu

_N = 21
_R = 64


def _cm_kernel(p_ref, t_ref, out_ref):
    ji = pl.program_id(1)

    x = p_ref[0]
    t = t_ref[0].reshape(1, _R * 512)
    iota = jax.lax.broadcasted_iota(jnp.int32, (_N, 1), 0)
    maxv = jnp.max(x, axis=0, keepdims=True)
    b_oh = (x == maxv).astype(jnp.bfloat16).reshape(_N, _R * 512)
    a_oh = (t == iota).astype(jnp.bfloat16)
    c = jax.lax.dot_general(
        a_oh, b_oh, (((1,), (1,)), ((), ())),
        preferred_element_type=jnp.float32)

    @pl.when(ji == 0)
    def _first():
        out_ref[0] = c

    @pl.when(ji != 0)
    def _rest():
        out_ref[0] += c


def _iou_eval_kernel(m_ref, out_ref):
    h = jnp.sum(m_ref[...], axis=0)
    r = jax.lax.broadcasted_iota(jnp.int32, (_N, _N), 0)
    cidx = jax.lax.broadcasted_iota(jnp.int32, (_N, _N), 1)
    eye = (r == cidx).astype(jnp.float32)
    ones = jnp.ones((_N, 1), jnp.float32)
    diag = jax.lax.dot_general(
        h * eye, ones, (((1,), (0,)), ((), ())),
        preferred_element_type=jnp.float32)
    rows = jax.lax.dot_general(
        h, ones, (((1,), (0,)), ((), ())),
        preferred_element_type=jnp.float32)
    cols = jax.lax.dot_general(
        h, ones, (((0,), (0,)), ((), ())),
        preferred_element_type=jnp.float32)
    iou = diag / (rows + cols - diag)
    out_ref[...] = (jnp.sum(iou) / _N).reshape(1, 1)


def kernel(preds, targets, mat):
    batch, n, hh, ww = preds.shape
    nb = hh // _R
    mats = pl.pallas_call(
        _cm_kernel,
        grid=(batch, nb),
        in_specs=[
            pl.BlockSpec((1, n, _R, ww), lambda b, j: (b, 0, j, 0)),
            pl.BlockSpec((1, _R, ww), lambda b, j: (b, j, 0)),
        ],
        out_specs=pl.BlockSpec((1, n, n), lambda b, j: (b, 0, 0)),
        out_shape=jax.ShapeDtypeStruct((batch, n, n), jnp.float32),
        compiler_params=pltpu.CompilerParams(
            dimension_semantics=("parallel", "arbitrary")),
    )(preds, targets)
    out = pl.pallas_call(
        _iou_eval_kernel,
        out_shape=jax.ShapeDtypeStruct((1, 1), jnp.float32),
    )(mats)
    return out[0, 0]
